# baseline (speedup 1.0000x reference)
import functools

import jax
import jax.numpy as jnp
from jax import lax
from jax.experimental import pallas as pl
from jax.experimental.pallas import tpu as pltpu
from jax.experimental.pallas import tpu_sc as plsc

NA = 10000
NP = 160000
RR = 128
NC = 2
NS = 16
PAD = 10000
TS = 640
PPS = NP // NC
K = 64
NCH = PPS // K
NTC = NCH // NS
NB = 2
NG = NTC // NB


_P_II = 2000


def _ii_body(i1_ref, w_ref, b_ref, out_ref):
    out_ref[...] = jnp.tanh(
        jnp.dot(i1_ref[...], w_ref[...], preferred_element_type=jnp.float32)
        + b_ref[...]
    )


def _ii_call(i1, w_ii, b_ii):
    grid = NP // _P_II
    return pl.pallas_call(
        _ii_body,
        grid=(grid,),
        in_specs=[
            pl.BlockSpec((_P_II, RR), lambda p: (p, 0)),
            pl.BlockSpec((RR, RR), lambda p: (0, 0)),
            pl.BlockSpec((1, RR), lambda p: (0, 0)),
        ],
        out_specs=pl.BlockSpec((_P_II, RR), lambda p: (p, 0)),
        out_shape=jax.ShapeDtypeStruct((NP, RR), jnp.float32),
    )(i1, w_ii, b_ii.reshape(1, RR))


def _sc_body(p3x_hbm, jjx_hbm, ii_hbm, i1g_hbm, d3t_hbm, out_hbm,
             j0, j1, q0, q1, rows_v, i1g_v, e0, e1,
             acc_sh, *sems):
    c = lax.axis_index("c")
    s = lax.axis_index("s")
    jidx_l = (j0, j1)
    iidx_l = (q0, q1)
    dxs_l = (e0, e1)
    ld_sems = sems[0:NB]
    g_sems = sems[NB:2 * NB]
    sc_sems = sems[2 * NB:3 * NB]

    def base_of(n):
        return c * PPS + (n * NS + s) * K

    def base2_of(n):
        return c * (PPS // 2) + (n * NS + s) * (K // 2)

    def loads(x, n, b):
        base = base_of(n)
        return (
            pltpu.async_copy(jjx_hbm.at[pl.ds(x * NP + base, K)],
                             jidx_l[b], ld_sems[b]),
            pltpu.async_copy(ii_hbm.at[pl.ds(base, K)],
                             iidx_l[b], ld_sems[b]),
            pltpu.async_copy(i1g_hbm.at[pl.ds(base2_of(n), K // 2)],
                             i1g_v.at[b], ld_sems[b]),
            pltpu.async_copy(d3t_hbm.at[pl.ds(x * NP + base, K)],
                             dxs_l[b].at[pl.ds(0, K)], ld_sems[b]),
        )

    def loads_wait(x, n, b):
        for d in loads_descs(x, n, b):
            d.wait()

    def loads_descs(x, n, b):
        base = base_of(n)
        return (
            pltpu.make_async_copy(jjx_hbm.at[pl.ds(x * NP + base, K)],
                                  jidx_l[b], ld_sems[b]),
            pltpu.make_async_copy(ii_hbm.at[pl.ds(base, K)],
                                  iidx_l[b], ld_sems[b]),
            pltpu.make_async_copy(i1g_hbm.at[pl.ds(base2_of(n), K // 2)],
                                  i1g_v.at[b], ld_sems[b]),
            pltpu.make_async_copy(d3t_hbm.at[pl.ds(x * NP + base, K)],
                                  dxs_l[b].at[pl.ds(0, K)], ld_sems[b]),
        )

    def gather(b):
        pltpu.async_copy(p3x_hbm.at[jidx_l[b]], rows_v.at[b], g_sems[b])

    def gather_wait(b):
        pltpu.make_async_copy(
            p3x_hbm.at[jidx_l[b]], rows_v.at[b], g_sems[b]
        ).wait()

    def scatter(b):
        pltpu.async_copy(rows_v.at[b], acc_sh.at[iidx_l[b]], sc_sems[b],
                         add=True)

    def scatter_wait(b):
        pltpu.make_async_copy(
            rows_v.at[b], acc_sh.at[iidx_l[b]], sc_sems[b]
        ).wait()

    def compute(b):
        sh = jnp.full((16,), 16, jnp.int32)
        msk = jnp.full((16,), -65536, jnp.int32)

        def _pair(k2, _):
            for u in range(2):
                k = k2 * 2 + u
                dx = dxs_l[b][pl.ds(k, 16)][0]
                for v in range(RR // 32):
                    w = i1g_v[b, k2, pl.ds(u * 64 + v * 16, 16)]
                    ga = jax.lax.bitcast_convert_type(w << sh, jnp.float32)
                    gb = jax.lax.bitcast_convert_type(w & msk, jnp.float32)
                    sa = pl.ds(v * 32, 16)
                    sb = pl.ds(v * 32 + 16, 16)
                    rows_v[b, k, sa] = (rows_v[b, k, sa] + dx) * ga
                    rows_v[b, k, sb] = (rows_v[b, k, sb] + dx) * gb
            return 0

        lax.fori_loop(0, K // 2, _pair, 0)

    for x in range(3):
        def _zero(k, _):
            rows_v[0, k // 8, pl.ds((k % 8) * 16, 16)] = jnp.zeros(
                (16,), jnp.float32
            )
            return 0

        lax.fori_loop(0, K * 8, _zero, 0)

        @pl.when(s < NS - 1)
        def _zfull():
            for t in range(TS // K):
                pltpu.sync_copy(rows_v.at[0],
                                acc_sh.at[pl.ds(s * TS + t * K, K)])

        @pl.when(s == NS - 1)
        def _zlast():
            for t in range(6):
                pltpu.sync_copy(rows_v.at[0],
                                acc_sh.at[pl.ds(s * TS + t * K, K)])
            pltpu.sync_copy(rows_v.at[0, pl.ds(0, 16)],
                            acc_sh.at[pl.ds(PAD - 16, 16)])

        plsc.subcore_barrier()

        for b in range(NB):
            loads(x, b, b)
        for b in range(NB):
            loads_wait(x, b, b)
            gather(b)

        @pl.loop(0, NG)
        def _group(g):
            for b in range(NB):
                gather_wait(b)
                compute(b)
                scatter(b)
            @pl.when(g < NG - 1)
            def _prefetch():
                for b in range(NB):
                    scatter_wait(b)
                    loads(x, g * NB + b + NB, b)
                for b in range(NB):
                    loads_wait(x, g * NB + b + NB, b)
                    gather(b)

        for b in range(NB):
            scatter_wait(b)

        @pl.when(s < 2)
        def _tail():
            base = c * PPS + (NCH - 2 + s) * K
            pltpu.sync_copy(jjx_hbm.at[pl.ds(x * NP + base, K)], jidx_l[0])
            pltpu.sync_copy(ii_hbm.at[pl.ds(base, K)], iidx_l[0])
            pltpu.async_copy(p3x_hbm.at[jidx_l[0]], rows_v.at[0],
                             g_sems[0])
            base2 = c * (PPS // 2) + (NCH - 2 + s) * (K // 2)
            pltpu.sync_copy(i1g_hbm.at[pl.ds(base2, K // 2)],
                            i1g_v.at[0])
            pltpu.sync_copy(d3t_hbm.at[pl.ds(x * NP + base, K)],
                            dxs_l[0].at[pl.ds(0, K)])
            gather_wait(0)
            compute(0)
            pltpu.sync_copy(rows_v.at[0], acc_sh.at[iidx_l[0]], add=True)

        plsc.subcore_barrier()

        @pl.when(s < NS - 1)
        def _dfull():
            pltpu.sync_copy(
                acc_sh.at[pl.ds(s * TS, TS)],
                out_hbm.at[c, x, pl.ds(s * TS, TS)],
            )

        @pl.when(s == NS - 1)
        def _dlast():
            pltpu.sync_copy(
                acc_sh.at[pl.ds(s * TS, PAD - (NS - 1) * TS)],
                out_hbm.at[c, x, pl.ds(s * TS, PAD - (NS - 1) * TS)],
            )

        plsc.subcore_barrier()


def _sc_call(p3x, jjx, ii, i1g, d3t):
    mesh = plsc.VectorSubcoreMesh(core_axis_name="c", subcore_axis_name="s")
    fn = functools.partial(
        pl.kernel,
        mesh=mesh,
        out_type=jax.ShapeDtypeStruct((NC, 3, PAD, RR), jnp.float32),
        scratch_types=(
            [pltpu.VMEM((K,), jnp.int32)] * NB
            + [pltpu.VMEM((K,), jnp.int32)] * NB
            + [
                pltpu.VMEM((NB, K, RR), jnp.float32),
                pltpu.VMEM((NB, K // 2, RR), jnp.int32),
            ]
            + [pltpu.VMEM((K + 16,), jnp.float32)] * NB
            + [pltpu.VMEM_SHARED((PAD, RR), jnp.float32)]
            + [pltpu.SemaphoreType.DMA] * (3 * NB)
        ),
    )(_sc_body)
    return fn(p3x, jjx, ii, i1g, d3t)


_B_PP = 400


def _pp_body(part_ref, w_ref, out_ref, dot_ref):
    acc = jnp.zeros((_B_PP, RR), jnp.float32)
    for x in range(3):
        px = jnp.dot(
            part_ref[0, x] + part_ref[1, x],
            w_ref[...],
            preferred_element_type=jnp.float32,
        )
        out_ref[:, x, :] = px
        acc = acc + px * px
    dot_ref[...] = acc


def _pp_call(part, w_pp):
    grid = NA // _B_PP
    return pl.pallas_call(
        _pp_body,
        grid=(grid,),
        in_specs=[
            pl.BlockSpec((NC, 3, _B_PP, RR), lambda p: (0, 0, p, 0)),
            pl.BlockSpec((RR, RR), lambda p: (0, 0)),
        ],
        out_specs=[
            pl.BlockSpec((_B_PP, 3, RR), lambda p: (p, 0, 0)),
            pl.BlockSpec((_B_PP, RR), lambda p: (p, 0)),
        ],
        out_shape=[
            jax.ShapeDtypeStruct((NA, 3, RR), jnp.float32),
            jax.ShapeDtypeStruct((NA, RR), jnp.float32),
        ],
    )(part, w_pp)


@jax.jit
def kernel(ind_2, p3, i1, d3, W_ii, b_ii, W_pp):
    ind_2 = ind_2.astype(jnp.int32)
    ii = ind_2[:, 0]
    jj = ind_2[:, 1]
    p3x = p3.reshape(3 * NA, RR)
    jjx = (3 * jj[None, :]
           + jnp.arange(3, dtype=jnp.int32)[:, None]).reshape(-1)
    d3t = d3.T.reshape(-1)

    v = jnp.arange(4)[:, None] * 32
    l = jnp.arange(16)[None, :]
    sig = jnp.stack([v + l, v + 16 + l], axis=-1).reshape(-1)
    i1g_f = _ii_call(i1, W_ii[:, sig], b_ii[sig])
    i1g = jax.lax.bitcast_convert_type(
        i1g_f.astype(jnp.bfloat16).reshape(NP, RR // 2, 2), jnp.int32
    ).reshape(NP // 2, RR)
    part = _sc_call(p3x, jjx, ii, i1g, d3t)
    p3_new, dotted = _pp_call(part, W_pp)
    return (p3_new, dotted)


# speedup vs baseline: 2.0696x; 2.0696x over previous
import functools

import jax
import jax.numpy as jnp
from jax import lax
from jax.experimental import pallas as pl
from jax.experimental.pallas import tpu as pltpu
from jax.experimental.pallas import tpu_sc as plsc

NA = 10000
NP = 160000
RR = 128
NC = 2
NS = 16
PAD = 10000
TS = 640
PPS = NP // NC
K = 64
NCH = PPS // K
NTC = NCH // NS
NB = 3
NG = NTC // NB


_P_II = 2000


def _ii_body(i1_ref, w_ref, b_ref, out_ref):
    out_ref[...] = jnp.tanh(
        jnp.dot(i1_ref[...], w_ref[...], preferred_element_type=jnp.float32)
        + b_ref[...]
    )


def _ii_call(i1, w_ii, b_ii):
    grid = NP // _P_II
    return pl.pallas_call(
        _ii_body,
        grid=(grid,),
        in_specs=[
            pl.BlockSpec((_P_II, RR), lambda p: (p, 0)),
            pl.BlockSpec((RR, RR), lambda p: (0, 0)),
            pl.BlockSpec((1, RR), lambda p: (0, 0)),
        ],
        out_specs=pl.BlockSpec((_P_II, RR), lambda p: (p, 0)),
        out_shape=jax.ShapeDtypeStruct((NP, RR), jnp.float32),
    )(i1, w_ii, b_ii.reshape(1, RR))


def _sc_body(p3x_hbm, jjx_hbm, ii_hbm, i1g_hbm, d3t_hbm, out_hbm,
             j0, j1, j2, q0, q1, q2, rows_v, i1g_v, e0, e1, e2,
             acc_sh, *sems):
    c = lax.axis_index("c")
    s = lax.axis_index("s")
    jidx_l = (j0, j1, j2)
    iidx_l = (q0, q1, q2)
    dxs_l = (e0, e1, e2)
    ld_sems = sems[0:NB]
    g_sems = sems[NB:2 * NB]
    sc_sems = sems[2 * NB:3 * NB]

    def base_of(n):
        return c * PPS + (n * NS + s) * K

    def loads(x, n, b):
        base = base_of(n)
        return (
            pltpu.async_copy(jjx_hbm.at[pl.ds(x * NP + base, K)],
                             jidx_l[b], ld_sems[b]),
            pltpu.async_copy(ii_hbm.at[pl.ds(base, K)],
                             iidx_l[b], ld_sems[b]),
            pltpu.async_copy(i1g_hbm.at[pl.ds(base, K)],
                             i1g_v.at[b], ld_sems[b]),
            pltpu.async_copy(d3t_hbm.at[pl.ds(x * NP + base, K)],
                             dxs_l[b].at[pl.ds(0, K)], ld_sems[b]),
        )

    def loads_wait(x, n, b):
        for d in loads_descs(x, n, b):
            d.wait()

    def loads_descs(x, n, b):
        base = base_of(n)
        return (
            pltpu.make_async_copy(jjx_hbm.at[pl.ds(x * NP + base, K)],
                                  jidx_l[b], ld_sems[b]),
            pltpu.make_async_copy(ii_hbm.at[pl.ds(base, K)],
                                  iidx_l[b], ld_sems[b]),
            pltpu.make_async_copy(i1g_hbm.at[pl.ds(base, K)],
                                  i1g_v.at[b], ld_sems[b]),
            pltpu.make_async_copy(d3t_hbm.at[pl.ds(x * NP + base, K)],
                                  dxs_l[b].at[pl.ds(0, K)], ld_sems[b]),
        )

    def gather(b):
        pltpu.async_copy(p3x_hbm.at[jidx_l[b]], rows_v.at[b], g_sems[b])

    def gather_wait(b):
        pltpu.make_async_copy(
            p3x_hbm.at[jidx_l[b]], rows_v.at[b], g_sems[b]
        ).wait()

    def scatter(b):
        pltpu.async_copy(rows_v.at[b], acc_sh.at[iidx_l[b]], sc_sems[b],
                         add=True)

    def scatter_wait(b):
        pltpu.make_async_copy(
            rows_v.at[b], acc_sh.at[iidx_l[b]], sc_sems[b]
        ).wait()

    def compute(b):
        def _pair(k2, _):
            for u in range(2):
                k = k2 * 2 + u
                dx = dxs_l[b][pl.ds(k, 16)][0]
                for v in range(RR // 16):
                    sl = pl.ds(v * 16, 16)
                    rows_v[b, k, sl] = (
                        rows_v[b, k, sl] + dx
                    ) * i1g_v[b, k, sl]
            return 0

        lax.fori_loop(0, K // 2, _pair, 0)

    for x in range(3):
        def _zero(k, _):
            rows_v[0, k // 8, pl.ds((k % 8) * 16, 16)] = jnp.zeros(
                (16,), jnp.float32
            )
            return 0

        lax.fori_loop(0, K * 8, _zero, 0)

        @pl.when(s < NS - 1)
        def _zfull():
            for t in range(TS // K):
                pltpu.sync_copy(rows_v.at[0],
                                acc_sh.at[pl.ds(s * TS + t * K, K)])

        @pl.when(s == NS - 1)
        def _zlast():
            for t in range(6):
                pltpu.sync_copy(rows_v.at[0],
                                acc_sh.at[pl.ds(s * TS + t * K, K)])
            pltpu.sync_copy(rows_v.at[0, pl.ds(0, 16)],
                            acc_sh.at[pl.ds(PAD - 16, 16)])

        plsc.subcore_barrier()

        for b in range(NB):
            loads(x, b, b)
        for b in range(NB):
            loads_wait(x, b, b)
            gather(b)

        @pl.loop(0, NG)
        def _group(g):
            for b in range(NB):
                gather_wait(b)
                compute(b)
                scatter(b)
            @pl.when(g < NG - 1)
            def _prefetch():
                for b in range(NB):
                    scatter_wait(b)
                    loads(x, g * NB + b + NB, b)
                for b in range(NB):
                    loads_wait(x, g * NB + b + NB, b)
                    gather(b)

        for b in range(NB):
            scatter_wait(b)

        @pl.when(s < 2)
        def _tail():
            base = c * PPS + (NCH - 2 + s) * K
            pltpu.sync_copy(jjx_hbm.at[pl.ds(x * NP + base, K)], jidx_l[0])
            pltpu.sync_copy(ii_hbm.at[pl.ds(base, K)], iidx_l[0])
            pltpu.async_copy(p3x_hbm.at[jidx_l[0]], rows_v.at[0],
                             g_sems[0])
            pltpu.sync_copy(i1g_hbm.at[pl.ds(base, K)], i1g_v.at[0])
            pltpu.sync_copy(d3t_hbm.at[pl.ds(x * NP + base, K)],
                            dxs_l[0].at[pl.ds(0, K)])
            gather_wait(0)
            compute(0)
            pltpu.sync_copy(rows_v.at[0], acc_sh.at[iidx_l[0]], add=True)

        plsc.subcore_barrier()

        @pl.when(s < NS - 1)
        def _dfull():
            pltpu.sync_copy(
                acc_sh.at[pl.ds(s * TS, TS)],
                out_hbm.at[c, x, pl.ds(s * TS, TS)],
            )

        @pl.when(s == NS - 1)
        def _dlast():
            pltpu.sync_copy(
                acc_sh.at[pl.ds(s * TS, PAD - (NS - 1) * TS)],
                out_hbm.at[c, x, pl.ds(s * TS, PAD - (NS - 1) * TS)],
            )

        plsc.subcore_barrier()


def _sc_call(p3x, jjx, ii, i1g, d3t):
    mesh = plsc.VectorSubcoreMesh(core_axis_name="c", subcore_axis_name="s")
    fn = functools.partial(
        pl.kernel,
        mesh=mesh,
        out_type=jax.ShapeDtypeStruct((NC, 3, PAD, RR), jnp.float32),
        scratch_types=(
            [pltpu.VMEM((K,), jnp.int32)] * NB
            + [pltpu.VMEM((K,), jnp.int32)] * NB
            + [
                pltpu.VMEM((NB, K, RR), jnp.float32),
                pltpu.VMEM((NB, K, RR), jnp.float32),
            ]
            + [pltpu.VMEM((K + 16,), jnp.float32)] * NB
            + [pltpu.VMEM_SHARED((PAD, RR), jnp.float32)]
            + [pltpu.SemaphoreType.DMA] * (3 * NB)
        ),
    )(_sc_body)
    return fn(p3x, jjx, ii, i1g, d3t)


_B_PP = 400


def _pp_body(part_ref, w_ref, out_ref, dot_ref):
    acc = jnp.zeros((_B_PP, RR), jnp.float32)
    for x in range(3):
        px = jnp.dot(
            part_ref[0, x] + part_ref[1, x],
            w_ref[...],
            preferred_element_type=jnp.float32,
        )
        out_ref[:, x, :] = px
        acc = acc + px * px
    dot_ref[...] = acc


def _pp_call(part, w_pp):
    grid = NA // _B_PP
    return pl.pallas_call(
        _pp_body,
        grid=(grid,),
        in_specs=[
            pl.BlockSpec((NC, 3, _B_PP, RR), lambda p: (0, 0, p, 0)),
            pl.BlockSpec((RR, RR), lambda p: (0, 0)),
        ],
        out_specs=[
            pl.BlockSpec((_B_PP, 3, RR), lambda p: (p, 0, 0)),
            pl.BlockSpec((_B_PP, RR), lambda p: (p, 0)),
        ],
        out_shape=[
            jax.ShapeDtypeStruct((NA, 3, RR), jnp.float32),
            jax.ShapeDtypeStruct((NA, RR), jnp.float32),
        ],
    )(part, w_pp)


@jax.jit
def kernel(ind_2, p3, i1, d3, W_ii, b_ii, W_pp):
    ind_2 = ind_2.astype(jnp.int32)
    ii = ind_2[:, 0]
    jj = ind_2[:, 1]
    p3x = p3.reshape(3 * NA, RR)
    jjx = (3 * jj[None, :]
           + jnp.arange(3, dtype=jnp.int32)[:, None]).reshape(-1)
    d3t = d3.T.reshape(-1)

    i1g = _ii_call(i1, W_ii, b_ii)
    part = _sc_call(p3x, jjx, ii, i1g, d3t)
    p3_new, dotted = _pp_call(part, W_pp)
    return (p3_new, dotted)


# speedup vs baseline: 2.4326x; 1.1754x over previous
import functools

import jax
import jax.numpy as jnp
from jax import lax
from jax.experimental import pallas as pl
from jax.experimental.pallas import tpu as pltpu
from jax.experimental.pallas import tpu_sc as plsc

NA = 10000
NP = 160000
RR = 128
NC = 2
NS = 16
PAD = 10000
TS = 640
PPS = NP // NC
K = 64
NCH = PPS // K
NTC = NCH // NS
NB = 3
NG = NTC // NB


_P_II = 2000


def _ii_body(i1_ref, w_ref, b_ref, out_ref):
    out_ref[...] = jnp.tanh(
        jnp.dot(i1_ref[...], w_ref[...], preferred_element_type=jnp.float32)
        + b_ref[...]
    )


def _ii_call(i1, w_ii, b_ii):
    grid = NP // _P_II
    return pl.pallas_call(
        _ii_body,
        grid=(grid,),
        in_specs=[
            pl.BlockSpec((_P_II, RR), lambda p: (p, 0)),
            pl.BlockSpec((RR, RR), lambda p: (0, 0)),
            pl.BlockSpec((1, RR), lambda p: (0, 0)),
        ],
        out_specs=pl.BlockSpec((_P_II, RR), lambda p: (p, 0)),
        out_shape=jax.ShapeDtypeStruct((NP, RR), jnp.float32),
    )(i1, w_ii, b_ii.reshape(1, RR))


def _sc_body(p3x_hbm, jjx_hbm, ii_hbm, i1g_hbm, d3t_hbm, out_hbm,
             j0, j1, j2, q0, q1, q2, rows_v, i1g_v, e0, e1, e2,
             acc_sh, *sems):
    c = lax.axis_index("c")
    s = lax.axis_index("s")
    jidx_l = (j0, j1, j2)
    iidx_l = (q0, q1, q2)
    dxs_l = (e0, e1, e2)
    ld_sems = sems[0:NB]
    g_sems = sems[NB:2 * NB]
    sc_sems = sems[2 * NB:3 * NB]

    def base_of(n):
        return c * PPS + (n * NS + s) * K

    def loads(x, n, b):
        base = base_of(n)
        return (
            pltpu.async_copy(jjx_hbm.at[pl.ds(x * NP + base, K)],
                             jidx_l[b], ld_sems[b]),
            pltpu.async_copy(ii_hbm.at[pl.ds(base, K)],
                             iidx_l[b], ld_sems[b]),
            pltpu.async_copy(i1g_hbm.at[pl.ds(base, K)],
                             i1g_v.at[b], ld_sems[b]),
            pltpu.async_copy(d3t_hbm.at[pl.ds(x * NP + base, K)],
                             dxs_l[b].at[pl.ds(0, K)], ld_sems[b]),
        )

    def loads_wait(x, n, b):
        for d in loads_descs(x, n, b):
            d.wait()

    def loads_descs(x, n, b):
        base = base_of(n)
        return (
            pltpu.make_async_copy(jjx_hbm.at[pl.ds(x * NP + base, K)],
                                  jidx_l[b], ld_sems[b]),
            pltpu.make_async_copy(ii_hbm.at[pl.ds(base, K)],
                                  iidx_l[b], ld_sems[b]),
            pltpu.make_async_copy(i1g_hbm.at[pl.ds(base, K)],
                                  i1g_v.at[b], ld_sems[b]),
            pltpu.make_async_copy(d3t_hbm.at[pl.ds(x * NP + base, K)],
                                  dxs_l[b].at[pl.ds(0, K)], ld_sems[b]),
        )

    def gather(b):
        pltpu.async_copy(p3x_hbm.at[jidx_l[b]], rows_v.at[b], g_sems[b])

    def gather_wait(b):
        pltpu.make_async_copy(
            p3x_hbm.at[jidx_l[b]], rows_v.at[b], g_sems[b]
        ).wait()

    def scatter(b):
        pltpu.async_copy(rows_v.at[b], acc_sh.at[iidx_l[b]], sc_sems[b],
                         add=True)

    def scatter_wait(b):
        pltpu.make_async_copy(
            rows_v.at[b], acc_sh.at[iidx_l[b]], sc_sems[b]
        ).wait()

    def compute(b):
        def _pair(k2, _):
            for u in range(2):
                k = k2 * 2 + u
                dx = dxs_l[b][pl.ds(k, 16)][0]
                for v in range(RR // 16):
                    sl = pl.ds(v * 16, 16)
                    rows_v[b, k, sl] = (
                        rows_v[b, k, sl] + dx
                    ) * i1g_v[b, k, sl]
            return 0

        lax.fori_loop(0, K // 2, _pair, 0)

    for x in range(3):
        def _zero(k, _):
            rows_v[0, k // 8, pl.ds((k % 8) * 16, 16)] = jnp.zeros(
                (16,), jnp.float32
            )
            return 0

        lax.fori_loop(0, K * 8, _zero, 0)

        @pl.when(s < NS - 1)
        def _zfull():
            for t in range(TS // K):
                pltpu.sync_copy(rows_v.at[0],
                                acc_sh.at[pl.ds(s * TS + t * K, K)])

        @pl.when(s == NS - 1)
        def _zlast():
            for t in range(6):
                pltpu.sync_copy(rows_v.at[0],
                                acc_sh.at[pl.ds(s * TS + t * K, K)])
            pltpu.sync_copy(rows_v.at[0, pl.ds(0, 16)],
                            acc_sh.at[pl.ds(PAD - 16, 16)])

        plsc.subcore_barrier()

        loads(x, 0, 0)
        loads(x, 1, 1)
        loads_wait(x, 0, 0)
        gather(0)

        @pl.loop(0, NG)
        def _group(g):
            for u in range(NB):
                b = u
                n = g * NB + u

                def _s1():
                    scatter_wait((u - 1) % NB)

                if u == 0:
                    pl.when(g > 0)(_s1)
                else:
                    _s1()

                def _s2():
                    loads(x, n + 2, (u + 2) % NB)

                if u == 0:
                    _s2()
                else:
                    pl.when(g < NG - 1)(_s2)

                def _s3():
                    loads_wait(x, n + 1, (u + 1) % NB)
                    gather((u + 1) % NB)

                if u == NB - 1:
                    pl.when(g < NG - 1)(_s3)
                else:
                    _s3()

                gather_wait(b)
                compute(b)
                scatter(b)

        scatter_wait((NTC - 1) % NB)

        @pl.when(s < 2)
        def _tail():
            base = c * PPS + (NCH - 2 + s) * K
            pltpu.sync_copy(jjx_hbm.at[pl.ds(x * NP + base, K)], jidx_l[0])
            pltpu.sync_copy(ii_hbm.at[pl.ds(base, K)], iidx_l[0])
            pltpu.async_copy(p3x_hbm.at[jidx_l[0]], rows_v.at[0],
                             g_sems[0])
            pltpu.sync_copy(i1g_hbm.at[pl.ds(base, K)], i1g_v.at[0])
            pltpu.sync_copy(d3t_hbm.at[pl.ds(x * NP + base, K)],
                            dxs_l[0].at[pl.ds(0, K)])
            gather_wait(0)
            compute(0)
            pltpu.sync_copy(rows_v.at[0], acc_sh.at[iidx_l[0]], add=True)

        plsc.subcore_barrier()

        @pl.when(s < NS - 1)
        def _dfull():
            pltpu.sync_copy(
                acc_sh.at[pl.ds(s * TS, TS)],
                out_hbm.at[c, x, pl.ds(s * TS, TS)],
            )

        @pl.when(s == NS - 1)
        def _dlast():
            pltpu.sync_copy(
                acc_sh.at[pl.ds(s * TS, PAD - (NS - 1) * TS)],
                out_hbm.at[c, x, pl.ds(s * TS, PAD - (NS - 1) * TS)],
            )

        plsc.subcore_barrier()


def _sc_call(p3x, jjx, ii, i1g, d3t):
    mesh = plsc.VectorSubcoreMesh(core_axis_name="c", subcore_axis_name="s")
    fn = functools.partial(
        pl.kernel,
        mesh=mesh,
        out_type=jax.ShapeDtypeStruct((NC, 3, PAD, RR), jnp.float32),
        scratch_types=(
            [pltpu.VMEM((K,), jnp.int32)] * NB
            + [pltpu.VMEM((K,), jnp.int32)] * NB
            + [
                pltpu.VMEM((NB, K, RR), jnp.float32),
                pltpu.VMEM((NB, K, RR), jnp.float32),
            ]
            + [pltpu.VMEM((K + 16,), jnp.float32)] * NB
            + [pltpu.VMEM_SHARED((PAD, RR), jnp.float32)]
            + [pltpu.SemaphoreType.DMA] * (3 * NB)
        ),
    )(_sc_body)
    return fn(p3x, jjx, ii, i1g, d3t)


_B_PP = 400


def _pp_body(part_ref, w_ref, out_ref, dot_ref):
    acc = jnp.zeros((_B_PP, RR), jnp.float32)
    for x in range(3):
        px = jnp.dot(
            part_ref[0, x] + part_ref[1, x],
            w_ref[...],
            preferred_element_type=jnp.float32,
        )
        out_ref[:, x, :] = px
        acc = acc + px * px
    dot_ref[...] = acc


def _pp_call(part, w_pp):
    grid = NA // _B_PP
    return pl.pallas_call(
        _pp_body,
        grid=(grid,),
        in_specs=[
            pl.BlockSpec((NC, 3, _B_PP, RR), lambda p: (0, 0, p, 0)),
            pl.BlockSpec((RR, RR), lambda p: (0, 0)),
        ],
        out_specs=[
            pl.BlockSpec((_B_PP, 3, RR), lambda p: (p, 0, 0)),
            pl.BlockSpec((_B_PP, RR), lambda p: (p, 0)),
        ],
        out_shape=[
            jax.ShapeDtypeStruct((NA, 3, RR), jnp.float32),
            jax.ShapeDtypeStruct((NA, RR), jnp.float32),
        ],
    )(part, w_pp)


@jax.jit
def kernel(ind_2, p3, i1, d3, W_ii, b_ii, W_pp):
    ind_2 = ind_2.astype(jnp.int32)
    ii = ind_2[:, 0]
    jj = ind_2[:, 1]
    p3x = p3.reshape(3 * NA, RR)
    jjx = (3 * jj[None, :]
           + jnp.arange(3, dtype=jnp.int32)[:, None]).reshape(-1)
    d3t = d3.T.reshape(-1)

    i1g = _ii_call(i1, W_ii, b_ii)
    part = _sc_call(p3x, jjx, ii, i1g, d3t)
    p3_new, dotted = _pp_call(part, W_pp)
    return (p3_new, dotted)
